# baseline (speedup 1.0000x reference)
import jax
import jax.numpy as jnp
from jax.experimental import pallas as pl
from jax.experimental.pallas import tpu as pltpu

_BIG = 3.0e38


def _body(p_ref, ev_ref, out_ref):
    mu = p_ref[0]
    t0 = p_ref[5]
    t1 = t0 + p_ref[6]
    t2 = t1 + p_ref[7]

    R = ev_ref.shape[0]

    lane = jax.lax.broadcasted_iota(jnp.int32, (1, 384), 1)
    m = lane % 3
    is0 = m == 0
    is1 = m == 1

    def per_field(v_tau, v_a, v_c):
        return jnp.where(is0, v_tau, jnp.where(is1, v_a, v_c))

    T0 = per_field(_BIG, t0, 1.5)
    T1 = per_field(_BIG, t1, 2.5)
    T2 = per_field(_BIG, t2, _BIG)
    M0 = per_field(1.0, 0.0, 0.0)
    B0 = per_field(0.0, p_ref[1], 0.0)
    C1 = per_field(0.0, p_ref[2], p_ref[8])
    C2 = per_field(0.0, p_ref[3], p_ref[9])
    C3 = per_field(0.0, p_ref[4], 0.0)

    x = ev_ref[...]
    y = jnp.where(x <= T0, M0 * x + B0,
        jnp.where(x <= T1, C1,
        jnp.where(x <= T2, C2, C3)))

    y1 = pltpu.roll(y, 383, 1)
    y2 = pltpu.roll(y, 382, 1)
    prod = y * y1 * y2

    row = jax.lax.broadcasted_iota(jnp.int32, (384, 128), 0)
    col = jax.lax.broadcasted_iota(jnp.int32, (384, 128), 1)
    sel = (row == 3 * col).astype(jnp.float32)
    r = jax.lax.dot_general(prod, sel, (((1,), (0,)), ((), ())),
                            preferred_element_type=jnp.float32)

    out_ref[...] = 1.0 - jnp.exp(-mu * r)


def kernel(events, params):
    n = events.shape[0]
    rows = n // 128
    ev = events.reshape(rows, 384)
    R = 512
    grid = (rows // R,)

    out = pl.pallas_call(
        _body,
        grid=grid,
        in_specs=[
            pl.BlockSpec(memory_space=pltpu.SMEM),
            pl.BlockSpec((R, 384), lambda i: (i, 0)),
        ],
        out_specs=pl.BlockSpec((R, 128), lambda i: (i, 0)),
        out_shape=jax.ShapeDtypeStruct((rows, 128), jnp.float32),
    )(params, ev)
    return out.reshape(n)


# speedup vs baseline: 46.8391x; 46.8391x over previous
import jax
import jax.numpy as jnp
from jax.experimental import pallas as pl
from jax.experimental.pallas import tpu as pltpu


def _body(p_ref, tau_ref, a_ref, c_ref, out_ref):
    mu = p_ref[0]
    t0 = p_ref[5]
    t1 = t0 + p_ref[6]
    t2 = t1 + p_ref[7]

    tau = tau_ref[...]
    a = a_ref[...]
    c = c_ref[...]

    f_ble = jnp.where(a <= t0, p_ref[1],
            jnp.where(a <= t1, p_ref[2],
            jnp.where(a <= t2, p_ref[3], p_ref[4])))
    f_con = jnp.where(c == 2.0, p_ref[8],
            jnp.where(c == 3.0, p_ref[9], 0.0))
    r = tau * f_ble * f_con
    out_ref[...] = 1.0 - jnp.exp(-mu * r)


def kernel(events, params):
    n = events.shape[0]
    rows = n // 128
    tau = events[:, 0].reshape(rows, 128)
    a = events[:, 1].reshape(rows, 128)
    c = events[:, 2].reshape(rows, 128)
    R = 1024
    grid = (rows // R,)

    spec = pl.BlockSpec((R, 128), lambda i: (i, 0))
    out = pl.pallas_call(
        _body,
        grid=grid,
        in_specs=[pl.BlockSpec(memory_space=pltpu.SMEM), spec, spec, spec],
        out_specs=spec,
        out_shape=jax.ShapeDtypeStruct((rows, 128), jnp.float32),
    )(params, tau, a, c)
    return out.reshape(n)
